# baseline (speedup 1.0000x reference)
import jax
import jax.numpy as jnp
from jax import lax
from jax.experimental import pallas as pl
from jax.experimental.pallas import tpu as pltpu
from jax.experimental.pallas import tpu_sc as plsc

NUM_BUCKETS = 1000000
DIM = 64
B_ROWS = 4
SEQ = 8192
TOTAL = B_ROWS * SEQ

_info = plsc.get_sparse_core_info()
NC, NS, L = _info.num_cores, _info.num_subcores, _info.num_lanes
NW = NC * NS
B_PER_W = TOTAL // NW
N_VEC = B_PER_W // 16

_DYN_GATHER_DNUMS = lax.GatherDimensionNumbers(
    offset_dims=(), collapsed_slice_dims=(0,), start_index_map=(0,)
)


def _lane(v, j):
    splat = lax.gather(
        v,
        jnp.full((16, 1), j, dtype=jnp.int32),
        _DYN_GATHER_DNUMS,
        (1,),
        mode=lax.GatherScatterMode.PROMISE_IN_BOUNDS,
    )
    return lax.reduce_max(splat, axes=(0,))


NSEM = 4
HALF = B_PER_W // 2


def _sc_kernel(ids_hbm, table_hbm, out_hbm, ext_v, rows_v, sems):
    wid = lax.axis_index("s") * NC + lax.axis_index("c")
    base = wid * B_PER_W

    pltpu.sync_copy(ids_hbm.at[pl.ds(base, B_PER_W + 8)], ext_v)

    lane = lax.iota(jnp.int32, 16)

    def make_group(p):
        def group(g, _):
            i0 = g * 16
            cur = ext_v[pl.ds(i0 + 8, 16)]
            prev = ext_v[pl.ds(i0 + 7, 16)]
            prev = prev * jnp.minimum((base + i0 + lane) & (SEQ - 1), 1)
            h = (prev * 31 + cur) % NUM_BUCKETS
            acc = 0
            for j in range(16):
                r = _lane(h, j)
                acc = acc + r
            pltpu.async_copy(
                table_hbm.at[pl.ds(jnp.minimum(acc, 0), 1)],
                rows_v.at[pl.ds(i0 - p * HALF, 1)],
                sems.at[0],
            )
            return 0

        return group

    for p in range(2):
        lax.fori_loop(
            p * (HALF // 16), (p + 1) * (HALF // 16), make_group(p), 0, unroll=2
        )
        pltpu.make_async_copy(
            table_hbm.at[pl.ds(0, HALF // 16)],
            rows_v.at[pl.ds(0, HALF // 16)],
            sems.at[0],
        ).wait()
        pltpu.sync_copy(rows_v, out_hbm.at[pl.ds(base + p * HALF, HALF)])


@jax.jit
def kernel(input_ids, emb_weight):
    ids_flat = input_ids.reshape(-1).astype(jnp.int32)
    ids_pad = jnp.concatenate([jnp.zeros((8,), jnp.int32), ids_flat])

    mesh = plsc.VectorSubcoreMesh(core_axis_name="c", subcore_axis_name="s")
    out = pl.kernel(
        _sc_kernel,
        mesh=mesh,
        out_type=jax.ShapeDtypeStruct((TOTAL, DIM), jnp.float32),
        scratch_types=[
            pltpu.VMEM((B_PER_W + 8,), jnp.int32),
            pltpu.VMEM((HALF, DIM), jnp.float32),
            pltpu.SemaphoreType.DMA((NSEM,)),
        ],
        compiler_params=pltpu.CompilerParams(
            use_tc_tiling_on_sc=True, needs_layout_passes=False
        ),
    )(ids_pad, emb_weight)
    return out.reshape(B_ROWS, SEQ, DIM)


# speedup vs baseline: 1.9369x; 1.9369x over previous
import jax
import jax.numpy as jnp
from jax import lax
from jax.experimental import pallas as pl
from jax.experimental.pallas import tpu as pltpu
from jax.experimental.pallas import tpu_sc as plsc

NUM_BUCKETS = 1000000
DIM = 64
B_ROWS = 4
SEQ = 8192
TOTAL = B_ROWS * SEQ

_info = plsc.get_sparse_core_info()
NC, NS, L = _info.num_cores, _info.num_subcores, _info.num_lanes
NW = NC * NS
B_PER_W = TOTAL // NW
N_VEC = B_PER_W // 16

def _lane(v, j):
    return v[j]


NSEM = 4
HALF = B_PER_W // 2


def _sc_kernel(ids_hbm, table_hbm, out_hbm, ext_v, rows_v, sems):
    wid = lax.axis_index("s") * NC + lax.axis_index("c")
    base = wid * B_PER_W

    pltpu.sync_copy(ids_hbm.at[pl.ds(base, B_PER_W + 8)], ext_v)

    lane = lax.iota(jnp.int32, 16)

    def make_group(p):
        def group(g, _):
            i0 = g * 16
            cur = ext_v[pl.ds(i0 + 8, 16)]
            prev = ext_v[pl.ds(i0 + 7, 16)]
            prev = prev * jnp.minimum((base + i0 + lane) & (SEQ - 1), 1)
            h = (prev * 31 + cur) % NUM_BUCKETS
            for j in range(16):
                r = _lane(h, j)
                pltpu.async_copy(
                    table_hbm.at[pl.ds(r, 1)],
                    rows_v.at[pl.ds(i0 - p * HALF + j, 1)],
                    sems.at[j % NSEM],
                )
            return 0

        return group

    for p in range(2):
        lax.fori_loop(
            p * (HALF // 16), (p + 1) * (HALF // 16), make_group(p), 0, unroll=2
        )
        for q in range(NSEM):
            pltpu.make_async_copy(
                table_hbm.at[pl.ds(0, HALF // NSEM)],
                rows_v.at[pl.ds(q * (HALF // NSEM), HALF // NSEM)],
                sems.at[q],
            ).wait()
        pltpu.sync_copy(rows_v, out_hbm.at[pl.ds(base + p * HALF, HALF)])


@jax.jit
def kernel(input_ids, emb_weight):
    ids_flat = input_ids.reshape(-1).astype(jnp.int32)
    ids_pad = jnp.concatenate([jnp.zeros((8,), jnp.int32), ids_flat])

    mesh = plsc.VectorSubcoreMesh(core_axis_name="c", subcore_axis_name="s")
    out = pl.kernel(
        _sc_kernel,
        mesh=mesh,
        out_type=jax.ShapeDtypeStruct((TOTAL, DIM), jnp.float32),
        scratch_types=[
            pltpu.VMEM((B_PER_W + 8,), jnp.int32),
            pltpu.VMEM((HALF, DIM), jnp.float32),
            pltpu.SemaphoreType.DMA((NSEM,)),
        ],
        compiler_params=pltpu.CompilerParams(
            use_tc_tiling_on_sc=True, needs_layout_passes=False
        ),
    )(ids_pad, emb_weight)
    return out.reshape(B_ROWS, SEQ, DIM)
